# baseline (speedup 1.0000x reference)
import functools

import jax
import jax.numpy as jnp
from jax import lax
from jax.experimental import pallas as pl
from jax.experimental.pallas import tpu as pltpu
from jax.experimental.pallas import tpu_sc as plsc

_BATCH = 16384
_VOCAB = 1000000
_D = 32
_L = 16
_NC = 2
_NS = 16
_NW = _NC * _NS
_BPW = _BATCH // _NW
_G = _BPW // _L
_RPW = _BPW * _D // 128
_NWIN = _VOCAB // 128
_TAIL0 = _NWIN * 128
_NTAIL = _VOCAB - _TAIL0
_TROWS = _NTAIL * _D // 128

_mesh = plsc.VectorSubcoreMesh(core_axis_name="c", subcore_axis_name="s")


@functools.partial(
    pl.kernel,
    mesh=_mesh,
    out_type=jax.ShapeDtypeStruct((_BATCH * _D // 128, 128), jnp.float32),
    scratch_types=[
        pltpu.VMEM((_BPW + _L,), jnp.int32),
        pltpu.VMEM((8, _D, 128), jnp.float32),
        pltpu.VMEM((8, _D, 128), jnp.float32),
        pltpu.VMEM((_TROWS, 128), jnp.float32),
        pltpu.VMEM((_TROWS, 128), jnp.float32),
        pltpu.VMEM((_RPW, 128), jnp.float32),
        pltpu.VMEM((_RPW, 128), jnp.float32),
        *[pltpu.SemaphoreType.DMA] * 17,
    ],
    compiler_params=pltpu.CompilerParams(use_tc_tiling_on_sc=True),
)
def _reparam_kernel(lab_hbm, mu_hbm, sg_hbm, mut_hbm, sgt_hbm, z_hbm, out_hbm,
                    idx_v, mu_s, sg_s, mut_v, sgt_v, z_v, out_v,
                    *sems):
    sems_mu = sems[0:8]
    sems_sg = sems[8:16]
    sem_z = sems[16]
    wid = lax.axis_index("s") * _NC + lax.axis_index("c")

    pltpu.sync_copy(lab_hbm.at[pl.ds(wid * _BPW, _BPW)],
                    idx_v.at[pl.ds(0, _BPW)])
    pltpu.sync_copy(mut_hbm, mut_v)
    pltpu.sync_copy(sgt_hbm, sgt_v)
    cp_z = pltpu.async_copy(z_hbm.at[pl.ds(wid * _RPW, _RPW)], z_v, sem_z)

    iota = lax.iota(jnp.int32, _L)
    onehots = [jnp.maximum(1 - jnp.abs(iota - c), 0).astype(jnp.float32)
               for c in range(_L)]

    def fire(r, buf):
        w = pl.multiple_of(
            jnp.minimum(lax.shift_right_logical(r, 7), _NWIN - 1) * 128, 128)
        pltpu.async_copy(mu_hbm.at[:, pl.ds(w, 128)], mu_s.at[buf],
                         sems_mu[buf])
        pltpu.async_copy(sg_hbm.at[:, pl.ds(w, 128)], sg_s.at[buf],
                         sems_sg[buf])

    def drain(buf):
        pltpu.make_async_copy(mu_hbm.at[:, pl.ds(0, 128)], mu_s.at[buf],
                              sems_mu[buf]).wait()
        pltpu.make_async_copy(sg_hbm.at[:, pl.ds(0, 128)], sg_s.at[buf],
                              sems_sg[buf]).wait()

    def label_at(j):
        return idx_v[pl.ds(j, _L)][0]

    for q in range(8):
        fire(label_at(q), q)
    cp_z.wait()

    def body(g, carry):
        for l in range(_L):
            p = l & 7
            j = g * _L + l
            drain(p)
            r = label_at(j)
            lane = r & 127
            tsel = jnp.where(r >= _TAIL0, 1.0, 0.0)
            tvec = iota * 0.0 + tsel
            tflat0 = jnp.maximum(r - _TAIL0, 0) * _D
            row_v = g * (_L // 4) + (l >> 2)
            for h in range(_D // _L):
                acc_m = jnp.zeros((_L,), jnp.float32)
                acc_s = jnp.zeros((_L,), jnp.float32)
                for c in range(_L):
                    cd = h * _L + c
                    oh = onehots[c]
                    vm = mu_s[p, cd, pl.ds(lane - c, _L)]
                    vs = sg_s[p, cd, pl.ds(lane - c, _L)]
                    tf = tflat0 + cd
                    trow = lax.shift_right_logical(tf, 7)
                    tcol = tf & 127
                    tvm = mut_v[trow, pl.ds(tcol - c, _L)]
                    tvs = sgt_v[trow, pl.ds(tcol - c, _L)]
                    acc_m = acc_m + oh * (vm + tvec * (tvm - vm))
                    acc_s = acc_s + oh * (vs + tvec * (tvs - vs))
                sl = pl.ds((l & 3) * _D + h * _L, _L)
                zz = z_v[row_v, sl]
                out_v[row_v, sl] = zz * jnp.exp(acc_s * 0.5) + acc_m
            fire(label_at(jnp.minimum(j + 8, _BPW - 1)), p)
        return carry

    lax.fori_loop(0, _G, body, 0)
    for q in range(8):
        drain(q)

    pltpu.sync_copy(out_v, out_hbm.at[pl.ds(wid * _RPW, _RPW)])


def kernel(labels, mu_table, sigma_table, z):
    lab = labels.astype(jnp.int32)
    mu_t = mu_table.T
    sg_t = sigma_table.T
    mu_tail = mu_table[_TAIL0:].reshape(_TROWS, 128)
    sg_tail = sigma_table[_TAIL0:].reshape(_TROWS, 128)
    z128 = z.reshape(_BATCH * _D // 128, 128)
    out128 = _reparam_kernel(lab, mu_t, sg_t, mu_tail, sg_tail, z128)
    return out128.reshape(_BATCH, _D)


# speedup vs baseline: 1.0876x; 1.0876x over previous
import functools

import jax
import jax.numpy as jnp
from jax import lax
from jax.experimental import pallas as pl
from jax.experimental.pallas import tpu as pltpu
from jax.experimental.pallas import tpu_sc as plsc

_BATCH = 16384
_VOCAB = 1000000
_D = 32
_L = 16
_NC = 2
_NS = 16
_NW = _NC * _NS
_BPW = _BATCH // _NW
_G = _BPW // _L
_RPW = _BPW * _D // 128
_NWIN = _VOCAB // 128
_TAIL0 = _NWIN * 128
_NTAIL = _VOCAB - _TAIL0
_TROWS = _NTAIL * _D // 128

_mesh = plsc.VectorSubcoreMesh(core_axis_name="c", subcore_axis_name="s")


@functools.partial(
    pl.kernel,
    mesh=_mesh,
    out_type=jax.ShapeDtypeStruct((_BATCH * _D // 128, 128), jnp.float32),
    scratch_types=[
        pltpu.VMEM((_BPW + _L,), jnp.int32),
        pltpu.VMEM((8, _D, 128), jnp.float32),
        pltpu.VMEM((8, _D, 128), jnp.float32),
        pltpu.VMEM((_TROWS, 128), jnp.float32),
        pltpu.VMEM((_TROWS, 128), jnp.float32),
        pltpu.VMEM((_RPW, 128), jnp.float32),
        pltpu.VMEM((_RPW, 128), jnp.float32),
        *[pltpu.SemaphoreType.DMA] * 17,
    ],
    compiler_params=pltpu.CompilerParams(use_tc_tiling_on_sc=True),
)
def _reparam_kernel(lab_hbm, mu_hbm, sg_hbm, mut_hbm, sgt_hbm, z_hbm, out_hbm,
                    idx_v, mu_s, sg_s, mut_v, sgt_v, z_v, out_v,
                    *sems):
    sems_mu = sems[0:8]
    sems_sg = sems[8:16]
    sem_z = sems[16]
    wid = lax.axis_index("s") * _NC + lax.axis_index("c")

    pltpu.sync_copy(lab_hbm.at[pl.ds(wid * _BPW, _BPW)],
                    idx_v.at[pl.ds(0, _BPW)])
    pltpu.sync_copy(mut_hbm, mut_v)
    pltpu.sync_copy(sgt_hbm, sgt_v)
    cp_z = pltpu.async_copy(z_hbm.at[pl.ds(wid * _RPW, _RPW)], z_v, sem_z)

    iota = lax.iota(jnp.int32, _L)
    onehots = [jnp.maximum(1 - jnp.abs(iota - c), 0).astype(jnp.float32)
               for c in range(_L)]

    def fire(r, buf):
        w = pl.multiple_of(
            jnp.minimum(lax.shift_right_logical(r, 7), _NWIN - 1) * 128, 128)
        pltpu.async_copy(mu_hbm.at[:, pl.ds(w, 128)], mu_s.at[buf],
                         sems_mu[buf])
        pltpu.async_copy(sg_hbm.at[:, pl.ds(w, 128)], sg_s.at[buf],
                         sems_sg[buf])

    def drain(buf):
        pltpu.make_async_copy(mu_hbm.at[:, pl.ds(0, 128)], mu_s.at[buf],
                              sems_mu[buf]).wait()
        pltpu.make_async_copy(sg_hbm.at[:, pl.ds(0, 128)], sg_s.at[buf],
                              sems_sg[buf]).wait()

    def label_at(j):
        return idx_v[pl.ds(j, _L)][0]

    for q in range(4):
        fire(label_at(q), q)
    cp_z.wait()

    def body(g, carry):
        for l in range(_L):
            p = l & 7
            j = g * _L + l
            rn = label_at(jnp.minimum(j + 4, _BPW - 1))
            fire(rn, (l + 4) & 7)
            drain(p)
            r = label_at(j)
            lane = r & 127
            tsel = jnp.where(r >= _TAIL0, 1.0, 0.0)
            tvec = iota * 0.0 + tsel
            tflat0 = jnp.maximum(r - _TAIL0, 0) * _D
            row_v = g * (_L // 4) + (l >> 2)
            for h in range(_D // _L):
                acc_m = jnp.zeros((_L,), jnp.float32)
                acc_s = jnp.zeros((_L,), jnp.float32)
                for c in range(_L):
                    cd = h * _L + c
                    oh = onehots[c]
                    vm = mu_s[p, cd, pl.ds(lane - c, _L)]
                    vs = sg_s[p, cd, pl.ds(lane - c, _L)]
                    tf = tflat0 + cd
                    trow = lax.shift_right_logical(tf, 7)
                    tcol = tf & 127
                    tvm = mut_v[trow, pl.ds(tcol - c, _L)]
                    tvs = sgt_v[trow, pl.ds(tcol - c, _L)]
                    acc_m = acc_m + oh * (vm + tvec * (tvm - vm))
                    acc_s = acc_s + oh * (vs + tvec * (tvs - vs))
                sl = pl.ds((l & 3) * _D + h * _L, _L)
                zz = z_v[row_v, sl]
                out_v[row_v, sl] = zz * jnp.exp(acc_s * 0.5) + acc_m
        return carry

    lax.fori_loop(0, _G, body, 0)
    for q in range(4):
        drain(q)

    pltpu.sync_copy(out_v, out_hbm.at[pl.ds(wid * _RPW, _RPW)])


def kernel(labels, mu_table, sigma_table, z):
    lab = labels.astype(jnp.int32)
    mu_t = mu_table.T
    sg_t = sigma_table.T
    mu_tail = mu_table[_TAIL0:].reshape(_TROWS, 128)
    sg_tail = sigma_table[_TAIL0:].reshape(_TROWS, 128)
    z128 = z.reshape(_BATCH * _D // 128, 128)
    out128 = _reparam_kernel(lab, mu_t, sg_t, mu_tail, sg_tail, z128)
    return out128.reshape(_BATCH, _D)
